# Initial kernel scaffold; baseline (speedup 1.0000x reference)
import jax
import jax.numpy as jnp
from jax.experimental import pallas as pl


def kernel(news_input, cat_input, ent_input, word_table, cat_table, ent_table, W, b):
    raise NotImplementedError("write your pallas kernel here")


# speedup vs baseline: 10.0696x; 10.0696x over previous
import functools

import jax
import jax.numpy as jnp
from jax import lax
from jax.experimental import pallas as pl
from jax.experimental.pallas import tpu as pltpu
from jax.experimental.pallas import tpu_sc as plsc

B = 16384
T = 50
D = 64
NC = 2
NS = 16
NW = NC * NS
RPW = B // NW
CHUNK = 16
NCHUNK = RPW // CHUNK


def _sc_gather_pool(news_flat, cat_idx, ent_idx, word_table, cat_table, ent_table):
  mesh = plsc.VectorSubcoreMesh(core_axis_name="c", subcore_axis_name="s")

  @functools.partial(
      pl.kernel,
      mesh=mesh,
      out_type=(
          jax.ShapeDtypeStruct((B, D), jnp.float32),
          jax.ShapeDtypeStruct((B, D), jnp.float32),
          jax.ShapeDtypeStruct((B, D), jnp.float32),
      ),
      compiler_params=pltpu.CompilerParams(use_tc_tiling_on_sc=False),
      scratch_types=[
          pltpu.VMEM((CHUNK * T,), jnp.int32),
          pltpu.VMEM((CHUNK * T, D), jnp.float32),
          pltpu.VMEM((CHUNK,), jnp.int32),
          pltpu.VMEM((CHUNK,), jnp.int32),
          pltpu.VMEM((CHUNK, D), jnp.float32),
          pltpu.VMEM((CHUNK, D), jnp.float32),
          pltpu.VMEM((CHUNK, D), jnp.float32),
          pltpu.SemaphoreType.DMA,
      ],
  )
  def body(news_r, cat_r, ent_r, wtab_r, ctab_r, etab_r,
           wsum_r, cvec_r, evec_r,
           idx_v, rows_v, cidx_v, eidx_v, crows_v, erows_v, acc_v, sem):
    wid = lax.axis_index("s") * NC + lax.axis_index("c")
    base = wid * RPW

    def chunk_body(j, carry):
      row0 = base + j * CHUNK
      pltpu.sync_copy(news_r.at[pl.ds(row0 * T, CHUNK * T)], idx_v)
      pltpu.sync_copy(cat_r.at[pl.ds(row0, CHUNK)], cidx_v)
      pltpu.sync_copy(ent_r.at[pl.ds(row0, CHUNK)], eidx_v)
      g1 = pltpu.async_copy(wtab_r.at[idx_v], rows_v, sem)
      g2 = pltpu.async_copy(ctab_r.at[cidx_v], crows_v, sem)
      g3 = pltpu.async_copy(etab_r.at[eidx_v], erows_v, sem)
      g1.wait()
      g2.wait()
      g3.wait()

      def row_body(r, rcarry):
        def t_body(t, accs):
          a0, a1, a2, a3 = accs
          src = r * T + t
          a0 = a0 + rows_v[src, 0:16]
          a1 = a1 + rows_v[src, 16:32]
          a2 = a2 + rows_v[src, 32:48]
          a3 = a3 + rows_v[src, 48:64]
          return (a0, a1, a2, a3)

        z = jnp.zeros((16,), jnp.float32)
        a0, a1, a2, a3 = lax.fori_loop(0, T, t_body, (z, z, z, z))
        acc_v[r, 0:16] = a0
        acc_v[r, 16:32] = a1
        acc_v[r, 32:48] = a2
        acc_v[r, 48:64] = a3
        return rcarry

      lax.fori_loop(0, CHUNK, row_body, 0)
      pltpu.sync_copy(acc_v, wsum_r.at[pl.ds(row0, CHUNK)])
      pltpu.sync_copy(crows_v, cvec_r.at[pl.ds(row0, CHUNK)])
      pltpu.sync_copy(erows_v, evec_r.at[pl.ds(row0, CHUNK)])
      return carry

    lax.fori_loop(0, NCHUNK, chunk_body, 0)

  return body(news_flat, cat_idx, ent_idx, word_table, cat_table, ent_table)


TC_BLK = 2048


def _tc_fuse(wsum, news, cvec, evec, W, b):
  def body(ws_r, news_r, cv_r, ev_r, w_r, b_r, out_r):
    mask = (news_r[...] != 0).astype(jnp.float32)
    cnt = jnp.sum(mask, axis=1, keepdims=True)
    wv = ws_r[...] / (cnt + 1e-08)
    dot = functools.partial(
        lax.dot_general,
        dimension_numbers=(((1,), (0,)), ((), ())),
        precision=lax.Precision.HIGHEST,
        preferred_element_type=jnp.float32,
    )
    acc = dot(wv, w_r[0:D, :])
    acc = acc + dot(cv_r[...], w_r[D:2 * D, :])
    acc = acc + dot(ev_r[...], w_r[2 * D:3 * D, :])
    out_r[...] = jnp.maximum(acc + b_r[...], 0.0)

  return pl.pallas_call(
      body,
      grid=(B // TC_BLK,),
      in_specs=[
          pl.BlockSpec((TC_BLK, D), lambda i: (i, 0)),
          pl.BlockSpec((TC_BLK, T), lambda i: (i, 0)),
          pl.BlockSpec((TC_BLK, D), lambda i: (i, 0)),
          pl.BlockSpec((TC_BLK, D), lambda i: (i, 0)),
          pl.BlockSpec((3 * D, D), lambda i: (0, 0)),
          pl.BlockSpec((1, D), lambda i: (0, 0)),
      ],
      out_specs=pl.BlockSpec((TC_BLK, D), lambda i: (i, 0)),
      out_shape=jax.ShapeDtypeStruct((B, D), jnp.float32),
  )(wsum, news, cvec, evec, W, b.reshape(1, D))


def kernel(news_input, cat_input, ent_input, word_table, cat_table, ent_table, W, b):
  news_input = news_input.astype(jnp.int32)
  cat_input = cat_input.astype(jnp.int32)
  ent_input = ent_input.astype(jnp.int32)
  news_flat = news_input.reshape(-1)
  wsum, cvec, evec = _sc_gather_pool(
      news_flat, cat_input, ent_input, word_table, cat_table, ent_table)
  return _tc_fuse(wsum, news_input, cvec, evec, W, b)


# speedup vs baseline: 15.2677x; 1.5162x over previous
import functools

import jax
import jax.numpy as jnp
from jax import lax
from jax.experimental import pallas as pl
from jax.experimental.pallas import tpu as pltpu
from jax.experimental.pallas import tpu_sc as plsc

B = 16384
T = 50
D = 64
NC = 2
NS = 16
NW = NC * NS
RPW = B // NW
CHUNK = 16
NCHUNK = RPW // CHUNK


def _sc_gather_pool(news_flat, cat_idx, ent_idx, word_table, cat_table, ent_table):
  mesh = plsc.VectorSubcoreMesh(core_axis_name="c", subcore_axis_name="s")

  buf = lambda shape, dtype: [pltpu.VMEM(shape, dtype)] * 2

  @functools.partial(
      pl.kernel,
      mesh=mesh,
      out_type=(
          jax.ShapeDtypeStruct((B, D), jnp.float32),
          jax.ShapeDtypeStruct((B, D), jnp.float32),
          jax.ShapeDtypeStruct((B, D), jnp.float32),
      ),
      compiler_params=pltpu.CompilerParams(use_tc_tiling_on_sc=False),
      scratch_types=[
          buf((CHUNK * T,), jnp.int32),
          buf((CHUNK * T, D), jnp.float32),
          buf((CHUNK,), jnp.int32),
          buf((CHUNK,), jnp.int32),
          buf((CHUNK, D), jnp.float32),
          buf((CHUNK, D), jnp.float32),
          buf((CHUNK, D), jnp.float32),
          [pltpu.SemaphoreType.DMA] * 2,
          [pltpu.SemaphoreType.DMA] * 2,
      ],
  )
  def body(news_r, cat_r, ent_r, wtab_r, ctab_r, etab_r,
           wsum_r, cvec_r, evec_r,
           idx_v, rows_v, cidx_v, eidx_v, crows_v, erows_v, acc_v,
           isem, gsem):
    wid = lax.axis_index("s") * NC + lax.axis_index("c")
    base = wid * RPW

    def start_idx(j, p):
      row0 = base + j * CHUNK
      pltpu.async_copy(news_r.at[pl.ds(row0 * T, CHUNK * T)], idx_v[p], isem[p])
      pltpu.async_copy(cat_r.at[pl.ds(row0, CHUNK)], cidx_v[p], isem[p])
      pltpu.async_copy(ent_r.at[pl.ds(row0, CHUNK)], eidx_v[p], isem[p])

    def wait_idx(p):
      pltpu.make_async_copy(news_r.at[pl.ds(0, CHUNK * T)], idx_v[p], isem[p]).wait()
      pltpu.make_async_copy(cat_r.at[pl.ds(0, CHUNK)], cidx_v[p], isem[p]).wait()
      pltpu.make_async_copy(ent_r.at[pl.ds(0, CHUNK)], eidx_v[p], isem[p]).wait()

    def fire_gathers(p):
      pltpu.async_copy(wtab_r.at[idx_v[p]], rows_v[p], gsem[p])
      pltpu.async_copy(ctab_r.at[cidx_v[p]], crows_v[p], gsem[p])
      pltpu.async_copy(etab_r.at[eidx_v[p]], erows_v[p], gsem[p])

    def wait_gathers(p):
      pltpu.make_async_copy(wtab_r.at[idx_v[p]], rows_v[p], gsem[p]).wait()
      pltpu.make_async_copy(ctab_r.at[cidx_v[p]], crows_v[p], gsem[p]).wait()
      pltpu.make_async_copy(etab_r.at[eidx_v[p]], erows_v[p], gsem[p]).wait()

    def compute_out(j, p):
      rows = rows_v[p]
      acc = acc_v[p]

      def row_body(r, rcarry):
        def t_body(t, accs):
          a0, a1, a2, a3 = accs
          src = r * T + t
          a0 = a0 + rows[src, 0:16]
          a1 = a1 + rows[src, 16:32]
          a2 = a2 + rows[src, 32:48]
          a3 = a3 + rows[src, 48:64]
          return (a0, a1, a2, a3)

        z = jnp.zeros((16,), jnp.float32)
        a0, a1, a2, a3 = lax.fori_loop(0, T, t_body, (z, z, z, z), unroll=5)
        acc[r, 0:16] = a0
        acc[r, 16:32] = a1
        acc[r, 32:48] = a2
        acc[r, 48:64] = a3
        return rcarry

      lax.fori_loop(0, CHUNK, row_body, 0)
      row0 = base + j * CHUNK
      pltpu.sync_copy(acc, wsum_r.at[pl.ds(row0, CHUNK)])
      pltpu.sync_copy(crows_v[p], cvec_r.at[pl.ds(row0, CHUNK)])
      pltpu.sync_copy(erows_v[p], evec_r.at[pl.ds(row0, CHUNK)])

    start_idx(0, 0)
    wait_idx(0)
    fire_gathers(0)
    start_idx(1, 1)

    def pair_body(jj, carry):
      j0 = 2 * jj
      j1 = j0 + 1
      wait_idx(1)
      fire_gathers(1)
      wait_gathers(0)

      @pl.when(j0 + 2 < NCHUNK)
      def _():
        start_idx(j0 + 2, 0)

      compute_out(j0, 0)

      @pl.when(j0 + 2 < NCHUNK)
      def _():
        wait_idx(0)
        fire_gathers(0)

      wait_gathers(1)

      @pl.when(j1 + 2 < NCHUNK)
      def _():
        start_idx(j1 + 2, 1)

      compute_out(j1, 1)
      return carry

    lax.fori_loop(0, NCHUNK // 2, pair_body, 0)

  return body(news_flat, cat_idx, ent_idx, word_table, cat_table, ent_table)


TC_BLK = 2048


def _tc_fuse(wsum, news, cvec, evec, W, b):
  def body(ws_r, news_r, cv_r, ev_r, w_r, b_r, out_r):
    mask = (news_r[...] != 0).astype(jnp.float32)
    cnt = jnp.sum(mask, axis=1, keepdims=True)
    wv = ws_r[...] / (cnt + 1e-08)
    dot = functools.partial(
        lax.dot_general,
        dimension_numbers=(((1,), (0,)), ((), ())),
        precision=lax.Precision.HIGHEST,
        preferred_element_type=jnp.float32,
    )
    acc = dot(wv, w_r[0:D, :])
    acc = acc + dot(cv_r[...], w_r[D:2 * D, :])
    acc = acc + dot(ev_r[...], w_r[2 * D:3 * D, :])
    out_r[...] = jnp.maximum(acc + b_r[...], 0.0)

  return pl.pallas_call(
      body,
      grid=(B // TC_BLK,),
      in_specs=[
          pl.BlockSpec((TC_BLK, D), lambda i: (i, 0)),
          pl.BlockSpec((TC_BLK, T), lambda i: (i, 0)),
          pl.BlockSpec((TC_BLK, D), lambda i: (i, 0)),
          pl.BlockSpec((TC_BLK, D), lambda i: (i, 0)),
          pl.BlockSpec((3 * D, D), lambda i: (0, 0)),
          pl.BlockSpec((1, D), lambda i: (0, 0)),
      ],
      out_specs=pl.BlockSpec((TC_BLK, D), lambda i: (i, 0)),
      out_shape=jax.ShapeDtypeStruct((B, D), jnp.float32),
  )(wsum, news, cvec, evec, W, b.reshape(1, D))


def kernel(news_input, cat_input, ent_input, word_table, cat_table, ent_table, W, b):
  news_input = news_input.astype(jnp.int32)
  cat_input = cat_input.astype(jnp.int32)
  ent_input = ent_input.astype(jnp.int32)
  news_flat = news_input.reshape(-1)
  wsum, cvec, evec = _sc_gather_pool(
      news_flat, cat_input, ent_input, word_table, cat_table, ent_table)
  return _tc_fuse(wsum, news_input, cvec, evec, W, b)
